# baseline (speedup 1.0000x reference)
import jax
import jax.numpy as jnp
from jax import lax
from jax.experimental import pallas as pl
from jax.experimental.pallas import tpu as pltpu
from jax.experimental.pallas import tpu_sc as plsc

_N = 262144
_NSC = 131072
_NTC = _N - _NSC
_NC = 2
_NS = 16
_NW = _NC * _NS
_L = 16
_A = _NSC // _NW
_ITERS = _A // _L
_U = 8
_NACC = 4

_C = (5.621959008883515e-07, 0.9999574870750662, -0.4992065685478449,
      0.32697310001386687, -0.2228362583280196, 0.13076503250423846,
      -0.052624851367851076, 0.010119082927824848)

_mesh = plsc.VectorSubcoreMesh(core_axis_name="c", subcore_axis_name="s")


def _sc_body(x0_hbm, x1_hbm, gt_hbm, out_hbm,
             x0_v, x1_v, gt_v, part_v, sem0, sem1, sem2):
    wid = lax.axis_index("s") * _NC + lax.axis_index("c")
    base = wid * _ITERS
    c0 = pltpu.async_copy(x0_hbm.at[pl.ds(base, _ITERS)], x0_v, sem0)
    c1 = pltpu.async_copy(x1_hbm.at[pl.ds(base, _ITERS)], x1_v, sem1)
    c2 = pltpu.async_copy(gt_hbm.at[pl.ds(base, _ITERS)], gt_v, sem2)
    c0.wait()
    c1.wait()
    c2.wait()

    zero = jnp.zeros((_L,), jnp.float32)

    def step(b, acc, cnt):
        y = gt_v[b]
        x0 = x0_v[b]
        x1 = x1_v[b]
        d = x1 - x0
        rp = jnp.maximum(d, 0.0)
        rm = rp - d
        pos = y == 1
        zrelu = jnp.where(pos, rm, rp)
        e = jnp.exp(jnp.minimum(d, -d))
        p = _C[7]
        for cf in _C[6::-1]:
            p = p * e + cf
        nll = zrelu + p
        valid = y != -1
        acc = acc + jnp.where(valid, nll, 0.0)
        cnt = cnt + jnp.where(valid, 1.0, 0.0)
        return acc, cnt

    def body(i, carry):
        accs = list(carry[:_NACC])
        cnts = list(carry[_NACC:])
        for k in range(_U):
            b = i * _U + k
            j = k % _NACC
            accs[j], cnts[j] = step(b, accs[j], cnts[j])
        return tuple(accs) + tuple(cnts)

    carry = plsc.parallel_loop(
        0, _ITERS // _U, carry=(zero,) * (2 * _NACC))(body)
    acc = carry[0] + carry[1] + carry[2] + carry[3]
    cnt = carry[4] + carry[5] + carry[6] + carry[7]

    part_v[pl.ds(0, _L)] = acc
    part_v[pl.ds(_L, _L)] = cnt
    pltpu.sync_copy(part_v, out_hbm.at[pl.ds(wid * 2 * _L, 2 * _L)])


def _tc_body(x0_ref, x1_ref, y_ref, o_ref):
    x0 = x0_ref[...]
    x1 = x1_ref[...]
    y = y_ref[...]
    d = x1 - x0
    rp = jnp.maximum(d, 0.0)
    zrelu = jnp.where(y == 1, rp - d, rp)
    nll = zrelu + jnp.log1p(jnp.exp(-jnp.abs(d)))
    m = (y != -1).astype(jnp.float32)
    o_ref[0, 0] = jnp.sum(nll * m)
    o_ref[0, 1] = jnp.sum(m)


def _tc_fin(p_ref, t_ref, o_ref):
    v = p_ref[...]
    k = lax.broadcasted_iota(jnp.int32, (8, 128), 1) % (2 * _L)
    is_sum = k < _L
    s = jnp.sum(jnp.where(is_sum, v, 0.0)) + t_ref[0, 0]
    c = jnp.sum(jnp.where(is_sum, 0.0, v)) + t_ref[0, 1]
    o_ref[0, 0] = jnp.clip(s / jnp.maximum(c, 1.0), 0.0, 10.0)


def kernel(pred_cls, gt_cls):
    x = pred_cls.reshape(_N, 2)
    gt = gt_cls.reshape(_N)
    x0s = x[:_NSC, 0].reshape(_NSC // _L, _L)
    x1s = x[:_NSC, 1].reshape(_NSC // _L, _L)
    gts = gt[:_NSC].reshape(_NSC // _L, _L)
    (xb,) = lax.optimization_barrier((x,))
    x0t = xb[_NSC:, 0].reshape(_NTC // 128, 128)
    x1t = xb[_NSC:, 1].reshape(_NTC // 128, 128)
    gtt = gt[_NSC:].reshape(_NTC // 128, 128)

    parts = pl.kernel(
        _sc_body,
        out_type=jax.ShapeDtypeStruct((_NW * 2 * _L,), jnp.float32),
        mesh=_mesh,
        scratch_types=[
            pltpu.VMEM((_ITERS, _L), jnp.float32),
            pltpu.VMEM((_ITERS, _L), jnp.float32),
            pltpu.VMEM((_ITERS, _L), jnp.int32),
            pltpu.VMEM((2 * _L,), jnp.float32),
            pltpu.SemaphoreType.DMA,
            pltpu.SemaphoreType.DMA,
            pltpu.SemaphoreType.DMA,
        ],
    )(x0s, x1s, gts)

    tc_part = pl.pallas_call(
        _tc_body,
        out_shape=jax.ShapeDtypeStruct((1, 2), jnp.float32),
        out_specs=pl.BlockSpec(memory_space=pltpu.SMEM),
    )(x0t, x1t, gtt)

    out = pl.pallas_call(
        _tc_fin,
        out_shape=jax.ShapeDtypeStruct((1, 1), jnp.float32),
        in_specs=[
            pl.BlockSpec((8, 128), lambda: (0, 0)),
            pl.BlockSpec(memory_space=pltpu.SMEM),
        ],
        out_specs=pl.BlockSpec(memory_space=pltpu.SMEM),
    )(parts.reshape(8, 128), tc_part)
    return out[0, 0]


# speedup vs baseline: 1.4239x; 1.4239x over previous
import jax
import jax.numpy as jnp
from jax import lax
from jax.experimental import pallas as pl
from jax.experimental.pallas import tpu as pltpu
from jax.experimental.pallas import tpu_sc as plsc

_N = 262144
_NSC = 131072
_NTC = _N - _NSC
_NC = 2
_NS = 16
_NW = _NC * _NS
_L = 16
_A = _NSC // _NW
_ITERS = _A // _L
_U = 16
_NACC = 8

_C = (5.621959008883515e-07, 0.9999574870750662, -0.4992065685478449,
      0.32697310001386687, -0.2228362583280196, 0.13076503250423846,
      -0.052624851367851076, 0.010119082927824848)

_mesh = plsc.VectorSubcoreMesh(core_axis_name="c", subcore_axis_name="s")


def _sc_body(x0_hbm, x1_hbm, gt_hbm, out_hbm,
             x0_v, x1_v, gt_v, part_v, sem0, sem1, sem2):
    wid = lax.axis_index("s") * _NC + lax.axis_index("c")
    base = wid * _A
    c0 = pltpu.async_copy(x0_hbm.at[pl.ds(base, _A)], x0_v, sem0)
    c1 = pltpu.async_copy(x1_hbm.at[pl.ds(base, _A)], x1_v, sem1)
    c2 = pltpu.async_copy(gt_hbm.at[pl.ds(base, _A)], gt_v, sem2)
    c0.wait()
    c1.wait()
    c2.wait()

    zero = jnp.zeros((_L,), jnp.float32)

    def step(b, acc, cnt):
        y = gt_v[pl.ds(b, _L)]
        x0 = x0_v[pl.ds(b, _L)]
        x1 = x1_v[pl.ds(b, _L)]
        d = x1 - x0
        rp = jnp.maximum(d, 0.0)
        rm = rp - d
        pos = y == 1
        zrelu = jnp.where(pos, rm, rp)
        e = jnp.exp(jnp.minimum(d, -d))
        p = _C[7]
        for cf in _C[6::-1]:
            p = p * e + cf
        nll = zrelu + p
        valid = y != -1
        acc = acc + jnp.where(valid, nll, 0.0)
        cnt = cnt + jnp.where(valid, 1.0, 0.0)
        return acc, cnt

    def body(i, carry):
        accs = list(carry[:_NACC])
        cnts = list(carry[_NACC:])
        for k in range(_U):
            b = (i * _U + k) * _L
            j = k % _NACC
            accs[j], cnts[j] = step(b, accs[j], cnts[j])
        return tuple(accs) + tuple(cnts)

    carry = plsc.parallel_loop(
        0, _ITERS // _U, carry=(zero,) * (2 * _NACC))(body)
    acc = sum(carry[1:_NACC], carry[0])
    cnt = sum(carry[_NACC + 1:], carry[_NACC])

    part_v[pl.ds(0, _L)] = acc
    part_v[pl.ds(_L, _L)] = cnt
    pltpu.sync_copy(part_v, out_hbm.at[pl.ds(wid * 2 * _L, 2 * _L)])


def _tc_body(x0_ref, x1_ref, y_ref, o_ref):
    x0 = x0_ref[...]
    x1 = x1_ref[...]
    y = y_ref[...]
    d = x1 - x0
    rp = jnp.maximum(d, 0.0)
    zrelu = jnp.where(y == 1, rp - d, rp)
    nll = zrelu + jnp.log1p(jnp.exp(-jnp.abs(d)))
    m = (y != -1).astype(jnp.float32)
    o_ref[0, 0] = jnp.sum(nll * m)
    o_ref[0, 1] = jnp.sum(m)


def _tc_fin(p_ref, t_ref, o_ref):
    v = p_ref[...]
    k = lax.broadcasted_iota(jnp.int32, (8, 128), 1) % (2 * _L)
    is_sum = k < _L
    s = jnp.sum(jnp.where(is_sum, v, 0.0)) + t_ref[0, 0]
    c = jnp.sum(jnp.where(is_sum, 0.0, v)) + t_ref[0, 1]
    o_ref[0, 0] = jnp.clip(s / jnp.maximum(c, 1.0), 0.0, 10.0)


def kernel(pred_cls, gt_cls):
    x = pred_cls.reshape(_N, 2)
    gt = gt_cls.reshape(_N)
    x0s = x[:_NSC, 0]
    x1s = x[:_NSC, 1]
    gts = gt[:_NSC]
    (xb,) = lax.optimization_barrier((x,))
    x0t = xb[_NSC:, 0].reshape(_NTC // 128, 128)
    x1t = xb[_NSC:, 1].reshape(_NTC // 128, 128)
    gtt = gt[_NSC:].reshape(_NTC // 128, 128)

    parts = pl.kernel(
        _sc_body,
        out_type=jax.ShapeDtypeStruct((_NW * 2 * _L,), jnp.float32),
        mesh=_mesh,
        scratch_types=[
            pltpu.VMEM((_A,), jnp.float32),
            pltpu.VMEM((_A,), jnp.float32),
            pltpu.VMEM((_A,), jnp.int32),
            pltpu.VMEM((2 * _L,), jnp.float32),
            pltpu.SemaphoreType.DMA,
            pltpu.SemaphoreType.DMA,
            pltpu.SemaphoreType.DMA,
        ],
    )(x0s, x1s, gts)

    tc_part = pl.pallas_call(
        _tc_body,
        out_shape=jax.ShapeDtypeStruct((1, 2), jnp.float32),
        out_specs=pl.BlockSpec(memory_space=pltpu.SMEM),
    )(x0t, x1t, gtt)

    out = pl.pallas_call(
        _tc_fin,
        out_shape=jax.ShapeDtypeStruct((1, 1), jnp.float32),
        in_specs=[
            pl.BlockSpec((8, 128), lambda: (0, 0)),
            pl.BlockSpec(memory_space=pltpu.SMEM),
        ],
        out_specs=pl.BlockSpec(memory_space=pltpu.SMEM),
    )(parts.reshape(8, 128), tc_part)
    return out[0, 0]
